# baseline (speedup 1.0000x reference)
import jax
import jax.numpy as jnp
from jax import lax
from jax.experimental import pallas as pl
import jax.experimental.pallas.tpu as pltpu
from jax.experimental.pallas import tpu_sc as plsc

SHAPE = (16, 32, 32)
D_PER = 80
EMBD = 240
NLANE = 16
NREG = D_PER // NLANE
HC = 2
NBUF = 2


def _sc_body(w0_hbm, w1_hbm, w2_hbm, out_hbm, bufs, w0v, w1v, w2v, sems):
    T, H, W = SHAPE
    B = out_hbm.shape[0]
    HH = H // 2
    n_rounds = HH // HC
    t = lax.axis_index("s")
    half = lax.axis_index("c")

    pltpu.sync_copy(w0_hbm.at[t], w0v)
    pltpu.sync_copy(w1_hbm.at[pl.ds(half * HH, HH)], w1v)
    pltpu.sync_copy(w2_hbm, w2v)

    w0regs = [w0v[pl.ds(k * NLANE, NLANE)] for k in range(NREG)]

    for j in range(n_rounds):
        buf = bufs.at[j % NBUF]
        if j >= NBUF:
            for b in range(B):
                pltpu.make_async_copy(
                    buf,
                    out_hbm.at[b, t, pl.ds(half * HH + (j - NBUF) * HC, HC)],
                    sems.at[j % NBUF],
                ).wait()

        w1regs = [
            [w1v[j * HC + h, pl.ds(k * NLANE, NLANE)] for k in range(NREG)]
            for h in range(HC)
        ]

        def w_body(w, _):
            for h in range(HC):
                for k in range(NREG):
                    buf[h, w, pl.ds(k * NLANE, NLANE)] = w0regs[k]
                for k in range(NREG):
                    buf[h, w, pl.ds(D_PER + k * NLANE, NLANE)] = w1regs[h][k]
                for k in range(NREG):
                    buf[h, w, pl.ds(2 * D_PER + k * NLANE, NLANE)] = w2v[
                        w, pl.ds(k * NLANE, NLANE)
                    ]
            return 0

        lax.fori_loop(0, W, w_body, 0)

        for b in range(B):
            pltpu.make_async_copy(
                buf, out_hbm.at[b, t, pl.ds(half * HH + j * HC, HC)], sems.at[j % NBUF]
            ).start()

    for j in range(n_rounds - NBUF, n_rounds):
        for b in range(B):
            pltpu.make_async_copy(
                bufs.at[j % NBUF],
                out_hbm.at[b, t, pl.ds(half * HH + j * HC, HC)],
                sems.at[j % NBUF],
            ).wait()


def kernel(x, W0, W1, W2):
    B = x.shape[0]
    T, H, W = SHAPE
    HH = H // 2
    run = pl.kernel(
        _sc_body,
        out_type=jax.ShapeDtypeStruct((B, T, H, W, EMBD), jnp.float32),
        mesh=plsc.VectorSubcoreMesh(core_axis_name="c", subcore_axis_name="s"),
        scratch_types=[
            pltpu.VMEM((NBUF, HC, W, EMBD), jnp.float32),
            pltpu.VMEM((D_PER,), jnp.float32),
            pltpu.VMEM((HH, D_PER), jnp.float32),
            pltpu.VMEM((W, D_PER), jnp.float32),
            pltpu.SemaphoreType.DMA((NBUF,)),
        ],
    )
    return run(W0, W1, W2)


# speedup vs baseline: 1.0145x; 1.0145x over previous
import jax
import jax.numpy as jnp
from jax import lax
from jax.experimental import pallas as pl
import jax.experimental.pallas.tpu as pltpu
from jax.experimental.pallas import tpu_sc as plsc

SHAPE = (16, 32, 32)
D_PER = 80
EMBD = 240
NLANE = 16
NREG = D_PER // NLANE
HC = 2
NBUF = 2


def _sc_body(w0_hbm, w1_hbm, w2_hbm, out_hbm, bufs, w0v, w1v, w2v, sems):
    T, H, W = SHAPE
    B = out_hbm.shape[0]
    HH = H // 2
    n_rounds = HH // HC
    t = lax.axis_index("s")
    half = lax.axis_index("c")

    cp0 = pltpu.make_async_copy(w0_hbm.at[t], w0v, sems.at[0])
    cp1 = pltpu.make_async_copy(w1_hbm.at[pl.ds(half * HH, HH)], w1v, sems.at[0])
    cp2 = pltpu.make_async_copy(w2_hbm, w2v, sems.at[1])
    cp0.start()
    cp1.start()
    cp2.start()
    cp0.wait()
    cp1.wait()
    cp2.wait()

    w0regs = [w0v[pl.ds(k * NLANE, NLANE)] for k in range(NREG)]

    for j in range(n_rounds):
        buf = bufs.at[j % NBUF]
        if j >= NBUF:
            for b in range(B):
                pltpu.make_async_copy(
                    buf,
                    out_hbm.at[b, t, pl.ds(half * HH + (j - NBUF) * HC, HC)],
                    sems.at[j % NBUF],
                ).wait()

        w1regs = [
            [w1v[j * HC + h, pl.ds(k * NLANE, NLANE)] for k in range(NREG)]
            for h in range(HC)
        ]

        def w_body(w, _):
            for h in range(HC):
                for k in range(NREG):
                    buf[h, w, pl.ds(k * NLANE, NLANE)] = w0regs[k]
                for k in range(NREG):
                    buf[h, w, pl.ds(D_PER + k * NLANE, NLANE)] = w1regs[h][k]
                for k in range(NREG):
                    buf[h, w, pl.ds(2 * D_PER + k * NLANE, NLANE)] = w2v[
                        w, pl.ds(k * NLANE, NLANE)
                    ]
            return 0

        lax.fori_loop(0, W, w_body, 0)

        for b in range(B):
            pltpu.make_async_copy(
                buf, out_hbm.at[b, t, pl.ds(half * HH + j * HC, HC)], sems.at[j % NBUF]
            ).start()

    for j in range(n_rounds - NBUF, n_rounds):
        for b in range(B):
            pltpu.make_async_copy(
                bufs.at[j % NBUF],
                out_hbm.at[b, t, pl.ds(half * HH + j * HC, HC)],
                sems.at[j % NBUF],
            ).wait()


def kernel(x, W0, W1, W2):
    B = x.shape[0]
    T, H, W = SHAPE
    HH = H // 2
    run = pl.kernel(
        _sc_body,
        out_type=jax.ShapeDtypeStruct((B, T, H, W, EMBD), jnp.float32),
        mesh=plsc.VectorSubcoreMesh(core_axis_name="c", subcore_axis_name="s"),
        scratch_types=[
            pltpu.VMEM((NBUF, HC, W, EMBD), jnp.float32),
            pltpu.VMEM((D_PER,), jnp.float32),
            pltpu.VMEM((HH, D_PER), jnp.float32),
            pltpu.VMEM((W, D_PER), jnp.float32),
            pltpu.SemaphoreType.DMA((NBUF,)),
        ],
    )
    return run(W0, W1, W2)
